# Initial kernel scaffold; baseline (speedup 1.0000x reference)
import jax
import jax.numpy as jnp
from jax.experimental import pallas as pl


def kernel(X, S, mask, chain_M, residue_idx, chain_encoding_all, decoding_order, params):
    raise NotImplementedError("write your pallas kernel here")


# speedup vs baseline: 4.2074x; 4.2074x over previous
import jax
import jax.numpy as jnp
from jax.experimental import pallas as pl

TL = 64
KNN = 32
H = 128
NRBF = 16
SCALE = 30.0


def _b_full(shape):
    n = len(shape)
    return pl.BlockSpec((1,) + shape, lambda b, t: (b,) + (0,) * n)


def _b_tile(shape):
    n = len(shape)
    return pl.BlockSpec((1, TL) + shape, lambda b, t: (b, t) + (0,) * n)


def _w(shape):
    n = len(shape)
    return pl.BlockSpec(shape, lambda b, t: (0,) * n)


def _ln(x, g_ref, b_ref):
    mu = jnp.mean(x, axis=-1, keepdims=True)
    xc = x - mu
    var = jnp.mean(xc * xc, axis=-1, keepdims=True)
    return xc * jax.lax.rsqrt(var + 1e-5) * g_ref[...] + b_ref[...]


def _onehot(idx, L):
    tl, kk = idx.shape
    iota = jax.lax.broadcasted_iota(jnp.int32, (tl, kk, L), 2)
    return (idx[:, :, None] == iota).astype(jnp.float32).reshape(tl * kk, L)


def _mm(a, b):
    return jnp.dot(a, b, preferred_element_type=jnp.float32)


def _feat_kernel(a5_ref, a5t_ref, aux_ref, auxt_ref, eidx_ref,
                 posw_ref, posb_ref, ew_ref, eb_ref, ng_ref, nbb_ref,
                 wew_ref, web_ref, he_out, att_out):
    L = a5_ref.shape[1]
    idx = eidx_ref[0]
    tl, kk = idx.shape
    oh = _onehot(idx, L)
    side = jnp.concatenate([a5_ref[0], aux_ref[0]], axis=-1)
    gat = _mm(oh, side).reshape(tl, kk, 18)
    xi = a5t_ref[0]
    mu = 2.0 + jax.lax.broadcasted_iota(jnp.int32, (1, 1, NRBF), 2).astype(jnp.float32) * (20.0 / (NRBF - 1))
    sig = 20.0 / NRBF
    feats = []
    for ai in range(5):
        pa = xi[:, 3 * ai:3 * ai + 3]
        for bi in range(5):
            pb = gat[:, :, 3 * bi:3 * bi + 3]
            d2 = jnp.sum((pa[:, None, :] - pb) ** 2, axis=-1, keepdims=True)
            dd = jnp.sqrt(d2 + 1e-6)
            feats.append(jnp.exp(-(((dd - mu) / sig) ** 2)))
    rbf = jnp.concatenate(feats, axis=-1).reshape(tl * kk, 25 * NRBF)
    auxt = auxt_ref[0]
    P = posw_ref.shape[0]
    maxrel = (P - 2) // 2
    ri = auxt[:, 0:1]
    rj = gat[:, :, 15:16]
    ci = auxt[:, 1:2]
    cj = gat[:, :, 16:17]
    ech = (jnp.abs(ci[:, None, :] - cj) < 0.5).astype(jnp.float32)
    off = ri[:, None, :] - rj
    d = jnp.clip(off + maxrel, 0.0, 2.0 * maxrel) * ech + (1.0 - ech) * (2.0 * maxrel + 1.0)
    iop = jax.lax.broadcasted_iota(jnp.int32, (tl, kk, P), 2).astype(jnp.float32)
    oh66 = (jnp.abs(d - iop) < 0.5).astype(jnp.float32).reshape(tl * kk, P)
    epos = _mm(oh66, posw_ref[...]) + posb_ref[...]
    e = jnp.concatenate([epos, rbf], axis=-1)
    e = _mm(e, ew_ref[...]) + eb_ref[...]
    e = _ln(e, ng_ref, nbb_ref)
    he = _mm(e, wew_ref[...]) + web_ref[...]
    he_out[0] = he.reshape(tl, kk, H)
    rk_i = auxt[:, 2:3]
    rk_j = gat[:, :, 17:18]
    att_out[0] = (rk_j < rk_i[:, None, :]).astype(jnp.float32)


def _enc_node_kernel(hv_ref, hvt_ref, he_ref, eidx_ref, mcol_ref, mt_ref,
                     w1, b1, w2, b2, w3, b3, g1, be1,
                     wi, bi, wo, bo, g2, be2, out_ref):
    L = hv_ref.shape[1]
    idx = eidx_ref[0]
    tl, kk = idx.shape
    oh = _onehot(idx, L)
    src = jnp.concatenate([hv_ref[0], mcol_ref[0]], axis=-1)
    g = _mm(oh, src).reshape(tl, kk, H + 1)
    hv_nb = g[:, :, :H]
    m_nb = g[:, :, H:H + 1]
    hvi = hvt_ref[0]
    hvb = jnp.broadcast_to(hvi[:, None, :], (tl, kk, H))
    hev = jnp.concatenate([hvb, he_ref[0], hv_nb], axis=-1).reshape(tl * kk, 3 * H)
    m = jax.nn.gelu(_mm(hev, w1[...]) + b1[...])
    m = jax.nn.gelu(_mm(m, w2[...]) + b2[...])
    m = (_mm(m, w3[...]) + b3[...]).reshape(tl, kk, H)
    ma = mt_ref[0][:, None, :] * m_nb
    dh = jnp.sum(m * ma, axis=1) / SCALE
    h = _ln(hvi + dh, g1, be1)
    f = jax.nn.gelu(_mm(h, wi[...]) + bi[...])
    h = _ln(h + _mm(f, wo[...]) + bo[...], g2, be2)
    out_ref[0] = h * mt_ref[0]


def _enc_edge_kernel(hv_ref, hvt_ref, he_ref, eidx_ref,
                     w1, b1, w2, b2, w3, b3, g3, be3, out_ref):
    L = hv_ref.shape[1]
    idx = eidx_ref[0]
    tl, kk = idx.shape
    oh = _onehot(idx, L)
    hv_nb = _mm(oh, hv_ref[0]).reshape(tl, kk, H)
    hvi = hvt_ref[0]
    hvb = jnp.broadcast_to(hvi[:, None, :], (tl, kk, H))
    hev = jnp.concatenate([hvb, he_ref[0], hv_nb], axis=-1).reshape(tl * kk, 3 * H)
    m = jax.nn.gelu(_mm(hev, w1[...]) + b1[...])
    m = jax.nn.gelu(_mm(m, w2[...]) + b2[...])
    m = (_mm(m, w3[...]) + b3[...]).reshape(tl, kk, H)
    out_ref[0] = _ln(he_ref[0] + m, g3, be3)


def _dec_kernel(hv_ref, hvt_ref, hvenc_ref, s_ref, he_ref, eidx_ref,
                att_ref, mt_ref, ws_ref,
                w1, b1, w2, b2, w3, b3, g1, be1,
                wi, bi, wo, bo, g2, be2, out_ref):
    L = hv_ref.shape[1]
    idx = eidx_ref[0]
    tl, kk = idx.shape
    oh = _onehot(idx, L)
    V = ws_ref.shape[0]
    iov = jax.lax.broadcasted_iota(jnp.int32, (L, V), 1)
    ohs = (s_ref[0] == iov).astype(jnp.float32)
    hs = _mm(ohs, ws_ref[...])
    src = jnp.concatenate([hv_ref[0], hvenc_ref[0], hs], axis=-1)
    g = _mm(oh, src).reshape(tl, kk, 3 * H)
    hv_nb = g[:, :, :H]
    hvenc_nb = g[:, :, H:2 * H]
    hs_nb = g[:, :, 2 * H:]
    att = att_ref[0]
    mi = mt_ref[0][:, None, :]
    bw = mi * att
    fw = mi * (1.0 - att)
    he = he_ref[0]
    hvi = hvt_ref[0]
    hvb = jnp.broadcast_to(hvi[:, None, :], (tl, kk, H))
    hev = jnp.concatenate([hvb,
                           he * (bw + fw),
                           hs_nb * bw,
                           hv_nb * bw + hvenc_nb * fw], axis=-1).reshape(tl * kk, 4 * H)
    m = jax.nn.gelu(_mm(hev, w1[...]) + b1[...])
    m = jax.nn.gelu(_mm(m, w2[...]) + b2[...])
    m = (_mm(m, w3[...]) + b3[...]).reshape(tl, kk, H)
    dh = jnp.sum(m, axis=1) / SCALE
    h = _ln(hvi + dh, g1, be1)
    f = jax.nn.gelu(_mm(h, wi[...]) + bi[...])
    h = _ln(h + _mm(f, wo[...]) + bo[...], g2, be2)
    out_ref[0] = h * mt_ref[0]


def _head_kernel(hvt_ref, mt_ref, wout_ref, bout_ref, out_ref):
    logits = (_mm(hvt_ref[0], wout_ref[...]) + bout_ref[...]) / 0.1
    mx = jnp.max(logits, axis=-1, keepdims=True)
    e = jnp.exp(logits - mx)
    p = e / jnp.sum(e, axis=-1, keepdims=True)
    out_ref[0] = jnp.log(p + 1e-20) * mt_ref[0]


def _lin2(p):
    return p["w"], p["b"].reshape(1, -1)


def _ln2(p):
    return p["g"].reshape(1, -1), p["b"].reshape(1, -1)


def kernel(X, S, mask, chain_M, residue_idx, chain_encoding_all,
           decoding_order, params):
    B, L = mask.shape
    f32 = jnp.float32
    Nb, Ca, Cc, O = X[:, :, 0], X[:, :, 1], X[:, :, 2], X[:, :, 3]
    bv = Ca - Nb
    cv = Cc - Ca
    av = jnp.cross(bv, cv)
    Cb = -0.58273431 * av + 0.56802827 * bv - 0.54067466 * cv + Ca
    A5 = jnp.concatenate([Ca, Nb, Cc, O, Cb], axis=-1)

    mask2D = mask[:, None, :] * mask[:, :, None]
    dX = Ca[:, None, :, :] - Ca[:, :, None, :]
    D = mask2D * jnp.sqrt(jnp.sum(dX ** 2, axis=-1) + 1e-6)
    D_max = jnp.max(D, axis=-1, keepdims=True)
    D_adjust = D + (1.0 - mask2D) * D_max
    _, E_idx = jax.lax.top_k(-D_adjust, KNN)

    ranks = jnp.argsort(decoding_order, axis=-1).astype(f32)
    aux = jnp.stack([residue_idx.astype(f32),
                     chain_encoding_all.astype(f32), ranks], axis=-1)
    mcol = mask[..., None]
    mask_eff = (chain_M * mask)
    del mask_eff

    grid = (B, L // TL)
    posw, posb = _lin2(params["pos_linear"])
    ew, eb = _lin2(params["edge_embedding"])
    ng, nbb = _ln2(params["norm_edges"])
    wew, web = _lin2(params["W_e"])

    h_E, att2 = pl.pallas_call(
        _feat_kernel,
        grid=grid,
        in_specs=[_b_full((L, 15)), _b_tile((15,)), _b_full((L, 3)),
                  _b_tile((3,)), _b_tile((KNN,)),
                  _w(posw.shape), _w(posb.shape), _w(ew.shape), _w(eb.shape),
                  _w(ng.shape), _w(nbb.shape), _w(wew.shape), _w(web.shape)],
        out_specs=[_b_tile((KNN, H)), _b_tile((KNN, 1))],
        out_shape=[jax.ShapeDtypeStruct((B, L, KNN, H), f32),
                   jax.ShapeDtypeStruct((B, L, KNN, 1), f32)],
    )(A5, A5, aux, aux, E_idx, posw, posb, ew, eb, ng, nbb, wew, web)

    h_V = jnp.zeros((B, L, H), dtype=f32)
    for lp in params["enc"]:
        w1, b1 = _lin2(lp["W1"])
        w2, b2 = _lin2(lp["W2"])
        w3, b3 = _lin2(lp["W3"])
        g1, be1 = _ln2(lp["norm1"])
        wi, bi = _lin2(lp["ffn_in"])
        wo, bo = _lin2(lp["ffn_out"])
        g2, be2 = _ln2(lp["norm2"])
        h_V = pl.pallas_call(
            _enc_node_kernel,
            grid=grid,
            in_specs=[_b_full((L, H)), _b_tile((H,)), _b_tile((KNN, H)),
                      _b_tile((KNN,)), _b_full((L, 1)), _b_tile((1,)),
                      _w(w1.shape), _w(b1.shape), _w(w2.shape), _w(b2.shape),
                      _w(w3.shape), _w(b3.shape), _w(g1.shape), _w(be1.shape),
                      _w(wi.shape), _w(bi.shape), _w(wo.shape), _w(bo.shape),
                      _w(g2.shape), _w(be2.shape)],
            out_specs=_b_tile((H,)),
            out_shape=jax.ShapeDtypeStruct((B, L, H), f32),
        )(h_V, h_V, h_E, E_idx, mcol, mcol, w1, b1, w2, b2, w3, b3,
          g1, be1, wi, bi, wo, bo, g2, be2)
        w11, b11 = _lin2(lp["W11"])
        w12, b12 = _lin2(lp["W12"])
        w13, b13 = _lin2(lp["W13"])
        g3, be3 = _ln2(lp["norm3"])
        h_E = pl.pallas_call(
            _enc_edge_kernel,
            grid=grid,
            in_specs=[_b_full((L, H)), _b_tile((H,)), _b_tile((KNN, H)),
                      _b_tile((KNN,)),
                      _w(w11.shape), _w(b11.shape), _w(w12.shape),
                      _w(b12.shape), _w(w13.shape), _w(b13.shape),
                      _w(g3.shape), _w(be3.shape)],
            out_specs=_b_tile((KNN, H)),
            out_shape=jax.ShapeDtypeStruct((B, L, KNN, H), f32),
        )(h_V, h_V, h_E, E_idx, w11, b11, w12, b12, w13, b13, g3, be3)

    h_V_enc = h_V
    S2 = S.astype(jnp.int32)[..., None]
    ws = params["W_s"]
    for lp in params["dec"]:
        w1, b1 = _lin2(lp["W1"])
        w2, b2 = _lin2(lp["W2"])
        w3, b3 = _lin2(lp["W3"])
        g1, be1 = _ln2(lp["norm1"])
        wi, bi = _lin2(lp["ffn_in"])
        wo, bo = _lin2(lp["ffn_out"])
        g2, be2 = _ln2(lp["norm2"])
        h_V = pl.pallas_call(
            _dec_kernel,
            grid=grid,
            in_specs=[_b_full((L, H)), _b_tile((H,)), _b_full((L, H)),
                      _b_full((L, 1)), _b_tile((KNN, H)), _b_tile((KNN,)),
                      _b_tile((KNN, 1)), _b_tile((1,)), _w(ws.shape),
                      _w(w1.shape), _w(b1.shape), _w(w2.shape), _w(b2.shape),
                      _w(w3.shape), _w(b3.shape), _w(g1.shape), _w(be1.shape),
                      _w(wi.shape), _w(bi.shape), _w(wo.shape), _w(bo.shape),
                      _w(g2.shape), _w(be2.shape)],
            out_specs=_b_tile((H,)),
            out_shape=jax.ShapeDtypeStruct((B, L, H), f32),
        )(h_V, h_V, h_V_enc, S2, h_E, E_idx, att2, mcol, ws,
          w1, b1, w2, b2, w3, b3, g1, be1, wi, bi, wo, bo, g2, be2)

    wout, bout = _lin2(params["W_out"])
    out = pl.pallas_call(
        _head_kernel,
        grid=grid,
        in_specs=[_b_tile((H,)), _b_tile((1,)),
                  _w(wout.shape), _w(bout.shape)],
        out_specs=_b_tile((wout.shape[1],)),
        out_shape=jax.ShapeDtypeStruct((B, L, wout.shape[1]), f32),
    )(h_V, mcol, wout, bout)
    return out
